# baseline (speedup 1.0000x reference)
import functools

import jax
import jax.numpy as jnp
from jax import lax
from jax.experimental import pallas as pl
from jax.experimental.pallas import tpu as pltpu
from jax.experimental.pallas import tpu_sc as plsc

BATCH = 16384
DIM = 16
NROWS = 1000000

_info = plsc.get_sparse_core_info()
_NC, _NS = _info.num_cores, _info.num_subcores
_NW = _NC * _NS
_BPW = BATCH // _NW
_CHUNK = 16
_NCHUNK = _BPW // _CHUNK

_mesh = plsc.VectorSubcoreMesh(core_axis_name="c", subcore_axis_name="s")


@functools.partial(
    pl.kernel,
    mesh=_mesh,
    compiler_params=pltpu.CompilerParams(needs_layout_passes=False),
    out_type=jax.ShapeDtypeStruct((BATCH,), jnp.float32),
    scratch_types=[
        pltpu.VMEM((_BPW,), jnp.int32),
        pltpu.VMEM((_BPW,), jnp.int32),
        pltpu.VMEM((DIM * DIM * 16,), jnp.float32),
        pltpu.VMEM((2, 8, _CHUNK * 128), jnp.float32),
        pltpu.VMEM((2, 8, _CHUNK * 128), jnp.float32),
        pltpu.VMEM((2, 8, _CHUNK * 128), jnp.float32),
        pltpu.VMEM((_BPW,), jnp.float32),
        pltpu.SemaphoreType.DMA,
        pltpu.SemaphoreType.DMA,
    ],
)
def _sc_svd(user_hbm, item_hbm, utab_hbm, itab_hbm, wsplat_hbm, out_hbm,
            uidx_v, iidx_v, w_v, ufa_v, ufb_v, if_v, out_v, sem_u, sem_i):
    wid = lax.axis_index("s") * _NC + lax.axis_index("c")
    base = wid * _BPW
    pltpu.sync_copy(user_hbm.at[pl.ds(base, _BPW)], uidx_v)
    pltpu.sync_copy(item_hbm.at[pl.ds(base, _BPW)], iidx_v)
    pltpu.sync_copy(wsplat_hbm, w_v)

    iota = lax.broadcasted_iota(jnp.int32, (16,), 0)

    def fire_user(c, ubuf):
        uvec_c = uidx_v[pl.ds(c * _CHUNK, 16)]

        @pl.loop(0, _CHUNK)
        def _fire(e):
            d128 = pl.ds(e * 128, 128)
            u = jnp.sum(jnp.where(iota == e, uvec_c, 0))
            us = pl.ds(pl.multiple_of((u >> 7) * 128, 128), 128)
            pltpu.async_copy(utab_hbm.at[:, :, us], ubuf.at[:, :, d128],
                             sem_u)

    def fire_item(c):
        ivec_c = iidx_v[pl.ds(c * _CHUNK, 16)]

        @pl.loop(0, _CHUNK)
        def _fire(e):
            d128 = pl.ds(e * 128, 128)
            i = jnp.sum(jnp.where(iota == e, ivec_c, 0))
            isl = pl.ds(pl.multiple_of((i >> 7) * 128, 128), 128)
            pltpu.async_copy(itab_hbm.at[:, :, isl], if_v.at[:, :, d128],
                             sem_i)

    def compute(c, ubuf):
        s16 = pl.ds(c * _CHUNK, 16)
        uvec = uidx_v[s16]
        ivec = iidx_v[s16]
        upos = iota * 128 + (uvec & 127)
        ipos = iota * 128 + (ivec & 127)
        zero = iota * 0
        ucols = [plsc.load_gather(ubuf, [zero + k // 8, zero + k % 8, upos])
                 for k in range(DIM)]
        icols = [plsc.load_gather(if_v, [zero + k // 8, zero + k % 8, ipos])
                 for k in range(DIM)]
        acc = jnp.zeros((16,), jnp.float32)
        for k in range(DIM):
            s = w_v[pl.ds(k * DIM * 16, 16)] * icols[0]
            for j in range(1, DIM):
                s = s + w_v[pl.ds((k * DIM + j) * 16, 16)] * icols[j]
            acc = acc + ucols[k] * s
        out_v[s16] = acc

    fire_user(0, ufa_v)
    fire_item(0)

    @pl.loop(0, _NCHUNK)
    def _chunk(c):
        pltpu.make_async_copy(
            utab_hbm.at[:, :, pl.ds(0, _CHUNK * 128)], ufa_v, sem_u).wait()
        pltpu.make_async_copy(
            utab_hbm.at[:, :, pl.ds(0, _CHUNK * 128)], if_v, sem_i).wait()

        @pl.when(c % 2 == 0)
        def _even():
            @pl.when(c < _NCHUNK - 1)
            def _pf():
                fire_user(c + 1, ufb_v)
            compute(c, ufa_v)

        @pl.when(c % 2 == 1)
        def _odd():
            @pl.when(c < _NCHUNK - 1)
            def _pf():
                fire_user(c + 1, ufa_v)
            compute(c, ufb_v)

        @pl.when(c < _NCHUNK - 1)
        def _pfi():
            fire_item(c + 1)

    pltpu.sync_copy(out_v, out_hbm.at[pl.ds(base, _BPW)])


@jax.jit
def kernel(user, item, user_table, item_table, svd_weight):
    user = user.astype(jnp.int32)
    item = item.astype(jnp.int32)
    ut3 = user_table.T.reshape(2, 8, NROWS)
    it3 = item_table.T.reshape(2, 8, NROWS)
    wsplat = jnp.repeat(svd_weight.reshape(DIM * DIM), 16)
    return _sc_svd(user, item, ut3, it3, wsplat)


# speedup vs baseline: 1.0193x; 1.0193x over previous
import functools

import jax
import jax.numpy as jnp
from jax import lax
from jax.experimental import pallas as pl
from jax.experimental.pallas import tpu as pltpu
from jax.experimental.pallas import tpu_sc as plsc

BATCH = 16384
DIM = 16
NROWS = 1000000

_info = plsc.get_sparse_core_info()
_NC, _NS = _info.num_cores, _info.num_subcores
_NW = _NC * _NS
_BPW = BATCH // _NW
_CHUNK = 16
_NCHUNK = _BPW // _CHUNK

_mesh = plsc.VectorSubcoreMesh(core_axis_name="c", subcore_axis_name="s")


@functools.partial(
    pl.kernel,
    mesh=_mesh,
    compiler_params=pltpu.CompilerParams(needs_layout_passes=False),
    out_type=jax.ShapeDtypeStruct((BATCH,), jnp.float32),
    scratch_types=[
        pltpu.VMEM((_BPW,), jnp.int32),
        pltpu.VMEM((_BPW,), jnp.int32),
        pltpu.VMEM((DIM * DIM * 16,), jnp.float32),
        pltpu.VMEM((2, 8, _CHUNK * 128), jnp.float32),
        pltpu.VMEM((2, 8, _CHUNK * 128), jnp.float32),
        pltpu.VMEM((_BPW,), jnp.float32),
        pltpu.SemaphoreType.DMA,
    ],
)
def _sc_svd(user_hbm, item_hbm, utab_hbm, itab_hbm, wsplat_hbm, out_hbm,
            uidx_v, iidx_v, w_v, uf_v, if_v, out_v, sem):
    wid = lax.axis_index("s") * _NC + lax.axis_index("c")
    base = wid * _BPW
    pltpu.sync_copy(user_hbm.at[pl.ds(base, _BPW)], uidx_v)
    pltpu.sync_copy(item_hbm.at[pl.ds(base, _BPW)], iidx_v)
    pltpu.sync_copy(wsplat_hbm, w_v)

    iota = lax.broadcasted_iota(jnp.int32, (16,), 0)

    @pl.loop(0, _NCHUNK)
    def _chunk(c):
        uvec_c = uidx_v[pl.ds(c * _CHUNK, 16)]
        ivec_c = iidx_v[pl.ds(c * _CHUNK, 16)]

        @pl.loop(0, _CHUNK)
        def _fire(e):
            d128 = pl.ds(e * 128, 128)
            u = jnp.sum(jnp.where(iota == e, uvec_c, 0))
            i = jnp.sum(jnp.where(iota == e, ivec_c, 0))
            us = pl.ds(pl.multiple_of((u >> 7) * 128, 128), 128)
            isl = pl.ds(pl.multiple_of((i >> 7) * 128, 128), 128)
            pltpu.async_copy(utab_hbm.at[:, :, us], uf_v.at[:, :, d128], sem)
            pltpu.async_copy(itab_hbm.at[:, :, isl], if_v.at[:, :, d128], sem)

        for buf in (uf_v, if_v):
            pltpu.make_async_copy(
                utab_hbm.at[:, :, pl.ds(0, _CHUNK * 128)], buf, sem).wait()

        s16 = pl.ds(c * _CHUNK, 16)
        uvec = uidx_v[s16]
        ivec = iidx_v[s16]
        upos = iota * 128 + (uvec & 127)
        ipos = iota * 128 + (ivec & 127)
        zero = iota * 0
        ucols = [plsc.load_gather(uf_v, [zero + k // 8, zero + k % 8, upos])
                 for k in range(DIM)]
        icols = [plsc.load_gather(if_v, [zero + k // 8, zero + k % 8, ipos])
                 for k in range(DIM)]
        acc = jnp.zeros((16,), jnp.float32)
        for k in range(DIM):
            s = w_v[pl.ds(k * DIM * 16, 16)] * icols[0]
            for j in range(1, DIM):
                s = s + w_v[pl.ds((k * DIM + j) * 16, 16)] * icols[j]
            acc = acc + ucols[k] * s
        out_v[s16] = acc

    pltpu.sync_copy(out_v, out_hbm.at[pl.ds(base, _BPW)])


@jax.jit
def kernel(user, item, user_table, item_table, svd_weight):
    user = user.astype(jnp.int32)
    item = item.astype(jnp.int32)
    ut3 = user_table.T.reshape(2, 8, NROWS)
    it3 = item_table.T.reshape(2, 8, NROWS)
    wsplat = jnp.repeat(svd_weight.reshape(DIM * DIM), 16)
    return _sc_svd(user, item, ut3, it3, wsplat)
